# baseline (speedup 1.0000x reference)
import functools

import jax
import jax.numpy as jnp
from jax import lax
from jax.experimental import pallas as pl
from jax.experimental.pallas import tpu as pltpu
from jax.experimental.pallas import tpu_sc as plsc

POI = 1000000
D = 64
B = 4096
H = 200
TOT = B * H
CH = 128
NBUF = 4

NC = 2
NS = 16
NW = NC * NS
NCH = TOT // (NW * CH)
NG = NCH // NBUF


@functools.lru_cache(maxsize=1)
def _build():
    mesh = plsc.VectorSubcoreMesh(core_axis_name="c", subcore_axis_name="s")

    @functools.partial(
        pl.kernel,
        mesh=mesh,
        out_type=jax.ShapeDtypeStruct((TOT, D), jnp.float32),
        compiler_params=pltpu.CompilerParams(use_tc_tiling_on_sc=False),
        scratch_types=(
            [pltpu.VMEM((NCH, CH), jnp.int32)]
            + [pltpu.VMEM((CH, D), jnp.float32) for _ in range(NBUF)]
            + [pltpu.SemaphoreType.DMA for _ in range(2 * NBUF)]
        ),
    )
    def gather_kernel(table_hbm, idx_hbm, out_hbm, idx_v, *rest):
        rows = rest[:NBUF]
        gsem = rest[NBUF:2 * NBUF]
        wsem = rest[2 * NBUF:]

        wid = lax.axis_index("s") * NC + lax.axis_index("c")
        pltpu.sync_copy(idx_hbm.at[pl.ds(wid * NCH, NCH)], idx_v)
        out_base = wid * NCH * CH

        for b in range(NBUF):
            pltpu.async_copy(table_hbm.at[idx_v.at[b]], rows[b], gsem[b])

        def group(g, carry):
            for b in range(NBUF):
                j = g * NBUF + b
                pltpu.make_async_copy(
                    table_hbm.at[idx_v.at[b]], rows[b], gsem[b]).wait()
                row0 = pl.multiple_of(out_base + j * CH, CH)
                pltpu.async_copy(
                    rows[b], out_hbm.at[pl.ds(row0, CH)], wsem[b])

                @pl.when(g < NG - 1)
                def _():
                    pltpu.make_async_copy(
                        rows[b], out_hbm.at[pl.ds(0, CH)], wsem[b]).wait()
                    pltpu.async_copy(
                        table_hbm.at[idx_v.at[j + NBUF]], rows[b], gsem[b])
            return carry

        lax.fori_loop(0, NG, group, 0)

        for b in range(NBUF):
            pltpu.make_async_copy(
                rows[b], out_hbm.at[pl.ds(0, CH)], wsem[b]).wait()

    return gather_kernel


def kernel(traj, table):
    flat_idx = traj.reshape(TOT // CH, CH).astype(jnp.int32)
    out = _build()(table, flat_idx)
    return out.reshape(B, H, D)
